# Initial kernel scaffold; baseline (speedup 1.0000x reference)
import functools

import jax
import jax.numpy as jnp
from jax import lax
from jax.experimental import pallas as pl
from jax.experimental.pallas import tpu as pltpu
from jax.experimental.pallas import tpu_sc as plsc

NC = 2
NS = 16
NW = NC * NS
LANES = 16


@functools.lru_cache(maxsize=None)
def _build(B, M, D):
    total = B * M
    per_w = total // NW
    assert per_w * NW == total
    reps = 8
    chunk = reps * M
    nchunk = per_w // chunk
    assert nchunk * chunk == per_w
    nvec = D // LANES
    assert nvec * LANES == D

    mesh = plsc.VectorSubcoreMesh(core_axis_name="c", subcore_axis_name="s")

    @functools.partial(
        pl.kernel,
        out_type=jax.ShapeDtypeStruct((total, D), jnp.float32),
        mesh=mesh,
        scratch_types=[
            pltpu.VMEM((2, chunk), jnp.int32),
            pltpu.VMEM((2, chunk, D), jnp.float32),
            pltpu.VMEM((M, D), jnp.float32),
            pltpu.SemaphoreType.DMA,
            pltpu.SemaphoreType.DMA,
            pltpu.SemaphoreType.DMA,
        ],
    )
    def k(x_hbm, tab_hbm, pos_hbm, out_hbm, idx_v, rows_v, pos_v,
          gsem0, gsem1, ssem):
        cid = lax.axis_index("c")
        sid = lax.axis_index("s")
        wid = sid * NC + cid
        base = wid * per_w

        pltpu.sync_copy(pos_hbm, pos_v)
        pltpu.sync_copy(x_hbm.at[pl.ds(base, chunk)], idx_v.at[0])
        gsems = (gsem0, gsem1)
        gd = [None, None]
        sd = [None] * nchunk
        gd[0] = pltpu.async_copy(tab_hbm.at[idx_v.at[0]], rows_v.at[0],
                                 gsems[0])

        for c in range(nchunk):
            b = c % 2
            nb = 1 - b
            if c + 1 < nchunk:
                pltpu.sync_copy(
                    x_hbm.at[pl.ds(base + (c + 1) * chunk, chunk)],
                    idx_v.at[nb])
                if c >= 1:
                    sd[c - 1].wait()
                gd[nb] = pltpu.async_copy(tab_hbm.at[idx_v.at[nb]],
                                          rows_v.at[nb], gsems[nb])
            gd[b].wait()

            def padd(p, carry, b=b):
                for v in range(nvec):
                    pv = pos_v[p, pl.ds(v * LANES, LANES)]
                    for rep in range(reps):
                        plsc.addupdate(
                            rows_v.at[b, rep * M + p,
                                      pl.ds(v * LANES, LANES)], pv)
                return carry

            lax.fori_loop(0, M, padd, None)
            sd[c] = pltpu.async_copy(
                rows_v.at[b],
                out_hbm.at[pl.ds(base + c * chunk, chunk)], ssem)
        sd[nchunk - 1].wait()

    return k


def kernel(x, token_table, pos_table):
    B, M = x.shape
    D = token_table.shape[1]
    k = _build(B, M, D)
    out = k(x.reshape(-1), token_table, pos_table)
    return out.reshape(B, M, D)


# speedup vs baseline: 1.4909x; 1.4909x over previous
import functools

import jax
import jax.numpy as jnp
from jax import lax
from jax.experimental import pallas as pl
from jax.experimental.pallas import tpu as pltpu
from jax.experimental.pallas import tpu_sc as plsc

NC = 2
NS = 16
NW = NC * NS
LANES = 16


@functools.lru_cache(maxsize=None)
def _build(B, M, D):
    total = B * M
    per_w = total // NW
    assert per_w * NW == total
    reps = 8
    chunk = reps * M
    nchunk = per_w // chunk
    assert nchunk * chunk == per_w
    nvec = D // LANES
    assert nvec * LANES == D

    mesh = plsc.VectorSubcoreMesh(core_axis_name="c", subcore_axis_name="s")

    @functools.partial(
        pl.kernel,
        out_type=jax.ShapeDtypeStruct((total, D), jnp.float32),
        mesh=mesh,
        compiler_params=pltpu.CompilerParams(use_tc_tiling_on_sc=False),
        scratch_types=[
            pltpu.VMEM((chunk,), jnp.int32),
            pltpu.VMEM((chunk,), jnp.int32),
            pltpu.VMEM((chunk, D), jnp.float32),
            pltpu.VMEM((chunk, D), jnp.float32),
            pltpu.VMEM((M, D), jnp.float32),
            pltpu.SemaphoreType.DMA,
            pltpu.SemaphoreType.DMA,
            pltpu.SemaphoreType.DMA,
        ],
    )
    def k(x_hbm, tab_hbm, pos_hbm, out_hbm, idx0, idx1, rows0, rows1,
          pos_v, gsem0, gsem1, ssem):
        cid = lax.axis_index("c")
        sid = lax.axis_index("s")
        wid = sid * NC + cid
        base = wid * per_w

        idx = (idx0, idx1)
        rows = (rows0, rows1)
        gsems = (gsem0, gsem1)

        pltpu.sync_copy(pos_hbm, pos_v)
        pltpu.sync_copy(x_hbm.at[pl.ds(base, chunk)], idx[0])
        gd = [None, None]
        sd = [None] * nchunk
        gd[0] = pltpu.async_copy(tab_hbm.at[idx[0]], rows[0], gsems[0])

        for c in range(nchunk):
            b = c % 2
            nb = 1 - b
            if c + 1 < nchunk:
                pltpu.sync_copy(
                    x_hbm.at[pl.ds(base + (c + 1) * chunk, chunk)],
                    idx[nb])
                if c >= 1:
                    sd[c - 1].wait()
                gd[nb] = pltpu.async_copy(tab_hbm.at[idx[nb]], rows[nb],
                                          gsems[nb])
            gd[b].wait()

            def padd(p, carry, b=b):
                for v in range(nvec):
                    pv = pos_v[p, pl.ds(v * LANES, LANES)]
                    for rep in range(reps):
                        plsc.addupdate(
                            rows[b].at[rep * M + p,
                                       pl.ds(v * LANES, LANES)], pv)
                return carry

            lax.fori_loop(0, M, padd, None)
            sd[c] = pltpu.async_copy(
                rows[b], out_hbm.at[pl.ds(base + c * chunk, chunk)], ssem)
        sd[nchunk - 1].wait()

    return k


def kernel(x, token_table, pos_table):
    B, M = x.shape
    D = token_table.shape[1]
    k = _build(B, M, D)
    out = k(x.reshape(-1), token_table, pos_table)
    return out.reshape(B, M, D)


# speedup vs baseline: 1.4914x; 1.0003x over previous
import functools

import jax
import jax.numpy as jnp
from jax import lax
from jax.experimental import pallas as pl
from jax.experimental.pallas import tpu as pltpu
from jax.experimental.pallas import tpu_sc as plsc

NC = 2
NS = 16
NW = NC * NS
LANES = 16


@functools.lru_cache(maxsize=None)
def _build(B, M, D):
    rows_w = B // NW
    assert rows_w * NW == B
    reps = 8
    chunk = reps * M
    nchunk = rows_w // reps
    assert nchunk * reps == rows_w
    nvec = D // LANES
    assert nvec * LANES == D

    mesh = plsc.VectorSubcoreMesh(core_axis_name="c", subcore_axis_name="s")

    @functools.partial(
        pl.kernel,
        out_type=jax.ShapeDtypeStruct((B, M, D), jnp.float32),
        mesh=mesh,
        compiler_params=pltpu.CompilerParams(use_tc_tiling_on_sc=False),
        scratch_types=[
            pltpu.VMEM((reps, M), jnp.int32),
            pltpu.VMEM((reps, M), jnp.int32),
            pltpu.VMEM((reps, M, D), jnp.float32),
            pltpu.VMEM((reps, M, D), jnp.float32),
            pltpu.VMEM((M, D), jnp.float32),
            pltpu.SemaphoreType.DMA,
            pltpu.SemaphoreType.DMA,
            pltpu.SemaphoreType.DMA,
        ],
    )
    def k(x_hbm, tab_hbm, pos_hbm, out_hbm, idx0, idx1, rows0, rows1,
          pos_v, gsem0, gsem1, ssem):
        cid = lax.axis_index("c")
        sid = lax.axis_index("s")
        wid = sid * NC + cid
        base = wid * rows_w

        idx = (idx0, idx1)
        rows = (rows0, rows1)
        gsems = (gsem0, gsem1)

        pltpu.sync_copy(pos_hbm, pos_v)
        pltpu.sync_copy(x_hbm.at[pl.ds(base, reps)], idx[0])
        gd = [None, None]
        sd = [None] * nchunk

        def start_gather(bb):
            return [pltpu.async_copy(tab_hbm.at[idx[bb].at[j]],
                                     rows[bb].at[j], gsems[bb])
                    for j in range(reps)]

        gd[0] = start_gather(0)

        for c in range(nchunk):
            b = c % 2
            nb = 1 - b
            if c + 1 < nchunk:
                pltpu.sync_copy(
                    x_hbm.at[pl.ds(base + (c + 1) * reps, reps)], idx[nb])
                if c >= 1:
                    sd[c - 1].wait()
                gd[nb] = start_gather(nb)
            for d in gd[b]:
                d.wait()

            def padd(p, carry, b=b):
                for v in range(nvec):
                    pv = pos_v[p, pl.ds(v * LANES, LANES)]
                    for rep in range(reps):
                        plsc.addupdate(
                            rows[b].at[rep, p, pl.ds(v * LANES, LANES)], pv)
                return carry

            lax.fori_loop(0, M, padd, None)
            sd[c] = pltpu.async_copy(
                rows[b], out_hbm.at[pl.ds(base + c * reps, reps)], ssem)
        sd[nchunk - 1].wait()

    return k


def kernel(x, token_table, pos_table):
    B, M = x.shape
    D = token_table.shape[1]
    k = _build(B, M, D)
    return k(x, token_table, pos_table)
